# baseline (speedup 1.0000x reference)
import functools

import jax
import jax.numpy as jnp
from jax import lax
from jax.experimental import pallas as pl
from jax.experimental.pallas import tpu as pltpu
from jax.experimental.pallas import tpu_sc as plsc


def _sc_gather(table, idx):
    V, D = table.shape
    N = idx.shape[0]
    rows_per_worker = 8
    n_workers = N // rows_per_worker
    mesh = plsc.VectorSubcoreMesh(core_axis_name="c", subcore_axis_name="s")

    @functools.partial(
        pl.kernel,
        mesh=mesh,
        out_type=jax.ShapeDtypeStruct((N, D), jnp.float32),
        scratch_types=[
            pltpu.VMEM((rows_per_worker,), jnp.int32),
            pltpu.VMEM((rows_per_worker, D), jnp.float32),
            pltpu.SemaphoreType.DMA,
        ],
    )
    def gather_kernel(table_hbm, idx_hbm, out_hbm, idx_v, rows_v, sem):
        info = plsc.get_sparse_core_info()
        wid = lax.axis_index("s") * info.num_cores + lax.axis_index("c")

        @pl.when(wid < n_workers)
        def _():
            base = wid * rows_per_worker
            pltpu.sync_copy(idx_hbm.at[pl.ds(base, rows_per_worker)], idx_v)
            pltpu.async_copy(table_hbm.at[idx_v], rows_v, sem).wait()
            pltpu.sync_copy(rows_v, out_hbm.at[pl.ds(base, rows_per_worker)])

    return gather_kernel(table, idx)


def _tc_add(x, emb):
    B, V, L, D = x.shape
    emb3 = emb.reshape(V, 1, D)

    def body(x_ref, e_ref, o_ref):
        o_ref[...] = x_ref[...] + e_ref[...]

    return pl.pallas_call(
        body,
        grid=(B, V),
        in_specs=[
            pl.BlockSpec((1, 1, L, D), lambda b, v: (b, v, 0, 0)),
            pl.BlockSpec((1, 1, D), lambda b, v: (v, 0, 0)),
        ],
        out_specs=pl.BlockSpec((1, 1, L, D), lambda b, v: (b, v, 0, 0)),
        out_shape=jax.ShapeDtypeStruct(x.shape, x.dtype),
    )(x, emb3)


def kernel(x, var_ids, var_embed):
    emb = _sc_gather(var_embed[0], var_ids)
    return _tc_add(x, emb)


# speedup vs baseline: 1.1468x; 1.1468x over previous
import jax
import jax.numpy as jnp
from jax.experimental import pallas as pl
from jax.experimental.pallas import tpu as pltpu


def kernel(x, var_ids, var_embed):
    B, V, L, D = x.shape
    emb3 = var_embed.reshape(V, 1, D)

    def body(ids_ref, x_ref, e_ref, o_ref):
        o_ref[...] = x_ref[...] + e_ref[...]

    grid_spec = pltpu.PrefetchScalarGridSpec(
        num_scalar_prefetch=1,
        grid=(B, V),
        in_specs=[
            pl.BlockSpec((1, 1, L, D), lambda b, v, ids: (b, v, 0, 0)),
            pl.BlockSpec((1, 1, D), lambda b, v, ids: (ids[v], 0, 0)),
        ],
        out_specs=pl.BlockSpec((1, 1, L, D), lambda b, v, ids: (b, v, 0, 0)),
    )
    return pl.pallas_call(
        body,
        grid_spec=grid_spec,
        out_shape=jax.ShapeDtypeStruct(x.shape, x.dtype),
    )(var_ids, x, emb3)


# speedup vs baseline: 1.2640x; 1.1023x over previous
import jax
import jax.numpy as jnp
from jax.experimental import pallas as pl
from jax.experimental.pallas import tpu as pltpu

_G = 4


def kernel(x, var_ids, var_embed):
    B, V, L, D = x.shape
    emb3 = var_embed.reshape(V, 1, D)

    def body(ids_ref, x_ref, e_ref, o_ref):
        vb = pl.program_id(1)
        rows = [e_ref[ids_ref[vb * _G + g]] for g in range(_G)]
        e = jnp.stack(rows, axis=0)
        o_ref[...] = x_ref[...] + e[None]

    grid_spec = pltpu.PrefetchScalarGridSpec(
        num_scalar_prefetch=1,
        grid=(B, V // _G),
        in_specs=[
            pl.BlockSpec((1, _G, L, D), lambda b, v, ids: (b, v, 0, 0)),
            pl.BlockSpec((V, 1, D), lambda b, v, ids: (0, 0, 0)),
        ],
        out_specs=pl.BlockSpec((1, _G, L, D), lambda b, v, ids: (b, v, 0, 0)),
    )
    return pl.pallas_call(
        body,
        grid_spec=grid_spec,
        out_shape=jax.ShapeDtypeStruct(x.shape, x.dtype),
    )(var_ids, x, emb3)
